# Initial kernel scaffold; baseline (speedup 1.0000x reference)
import functools

import jax
import jax.numpy as jnp
from jax import lax
from jax.experimental import pallas as pl
from jax.experimental.pallas import tpu as pltpu
from jax.experimental.pallas import tpu_sc as plsc

N_NODES = 10000
N_EDGES = 320000
D = 128

NC = 2
NS = 16
LANES = 16

EDGES_PER_CORE = N_EDGES // NC
EDGES_PER_TILE = EDGES_PER_CORE // NS
CHUNK = 80
N_CHUNKS = EDGES_PER_TILE // CHUNK
ROWS_PER_TILE = N_NODES // NS
CNT_W = LANES

_mesh = plsc.VectorSubcoreMesh(core_axis_name="c", subcore_axis_name="s")


@functools.partial(
    pl.kernel,
    out_type=(
        jax.ShapeDtypeStruct((NC, N_NODES, D), jnp.float32),
        jax.ShapeDtypeStruct((NC, N_NODES, CNT_W), jnp.float32),
    ),
    mesh=_mesh,
    scratch_types=(
        pltpu.VMEM_SHARED((N_NODES, D), jnp.float32),
        pltpu.VMEM_SHARED((N_NODES, CNT_W), jnp.float32),
        pltpu.VMEM((ROWS_PER_TILE, D), jnp.float32),
        pltpu.VMEM((ROWS_PER_TILE, CNT_W), jnp.float32),
        pltpu.VMEM((CHUNK, CNT_W), jnp.float32),
        pltpu.VMEM((CHUNK,), jnp.int32),
        pltpu.VMEM((CHUNK,), jnp.int32),
        pltpu.VMEM((CHUNK, D), jnp.float32),
        pltpu.SemaphoreType.DMA,
    ),
)
def _sc_aggregate(x_hbm, src_hbm, dst_hbm, acc_out, cnt_out,
                  acc_sh, cnt_sh, zrow, zcnt, ones_b, idx_s, idx_d, rows, sem):
    c = lax.axis_index("c")
    s = lax.axis_index("s")

    zero16 = jnp.zeros((LANES,), jnp.float32)
    one16 = jnp.ones((LANES,), jnp.float32)

    def fill_zero(i, carry):
        for j in range(D // LANES):
            zrow[i, pl.ds(j * LANES, LANES)] = zero16
        zcnt[i, :] = zero16
        return carry

    lax.fori_loop(0, ROWS_PER_TILE, fill_zero, 0)

    def fill_one(i, carry):
        ones_b[i, :] = one16
        return carry

    lax.fori_loop(0, CHUNK, fill_one, 0)

    row0 = s * ROWS_PER_TILE
    pltpu.sync_copy(zrow, acc_sh.at[pl.ds(row0, ROWS_PER_TILE)])
    pltpu.sync_copy(zcnt, cnt_sh.at[pl.ds(row0, ROWS_PER_TILE)])
    plsc.subcore_barrier()

    ebase = c * EDGES_PER_CORE + s * EDGES_PER_TILE

    def step(i, carry):
        off = pl.multiple_of(ebase + i * CHUNK, 8)
        pltpu.sync_copy(src_hbm.at[pl.ds(off, CHUNK)], idx_s)
        pltpu.sync_copy(dst_hbm.at[pl.ds(off, CHUNK)], idx_d)
        pltpu.async_copy(x_hbm.at[idx_s], rows, sem).wait()
        pltpu.sync_copy(rows, acc_sh.at[idx_d], add=True)
        pltpu.sync_copy(ones_b, cnt_sh.at[idx_d], add=True)
        return carry

    lax.fori_loop(0, N_CHUNKS, step, 0)
    plsc.subcore_barrier()

    pltpu.sync_copy(acc_sh.at[pl.ds(row0, ROWS_PER_TILE)], zrow)
    pltpu.sync_copy(zrow, acc_out.at[c, pl.ds(row0, ROWS_PER_TILE)])
    pltpu.sync_copy(cnt_sh.at[pl.ds(row0, ROWS_PER_TILE)], zcnt)
    pltpu.sync_copy(zcnt, cnt_out.at[c, pl.ds(row0, ROWS_PER_TILE)])


_BR = 400


def _tc_body(p_ref, cnt_ref, x_ref, wl_ref, wr_ref, b_ref, o_ref):
    p = p_ref[0] + p_ref[1]
    cnt = cnt_ref[0] + cnt_ref[1]
    scale = 1.0 / jnp.maximum(cnt, 1.0)
    aggr = p * scale[:, None]
    dn = (((1,), (1,)), ((), ()))
    o_ref[...] = (
        lax.dot_general(aggr, wl_ref[...], dn, preferred_element_type=jnp.float32)
        + lax.dot_general(x_ref[...], wr_ref[...], dn, preferred_element_type=jnp.float32)
        + b_ref[...]
    )


def _tc_finish(p, cnt, x, W_l, W_r, b):
    grid = (N_NODES // _BR,)
    return pl.pallas_call(
        _tc_body,
        grid=grid,
        in_specs=[
            pl.BlockSpec((NC, _BR, D), lambda i: (0, i, 0)),
            pl.BlockSpec((NC, _BR), lambda i: (0, i)),
            pl.BlockSpec((_BR, D), lambda i: (i, 0)),
            pl.BlockSpec((D, D), lambda i: (0, 0)),
            pl.BlockSpec((D, D), lambda i: (0, 0)),
            pl.BlockSpec((1, D), lambda i: (0, 0)),
        ],
        out_specs=pl.BlockSpec((_BR, D), lambda i: (i, 0)),
        out_shape=jax.ShapeDtypeStruct((N_NODES, D), jnp.float32),
    )(p, cnt, x, W_l, W_r, b)


def kernel(x, edge_index, W_l, W_r, b):
    src = edge_index[0]
    dst = edge_index[1]
    p, cnt_w = _sc_aggregate(x, src, dst)
    cnt = cnt_w[:, :, 0]
    return _tc_finish(p, cnt, x, W_l, W_r, b.reshape(1, D))


# speedup vs baseline: 3.5565x; 3.5565x over previous
import functools

import jax
import jax.numpy as jnp
from jax import lax
from jax.experimental import pallas as pl
from jax.experimental.pallas import tpu as pltpu
from jax.experimental.pallas import tpu_sc as plsc

N_NODES = 10000
N_EDGES = 320000
D = 128

NC = 2
NS = 16
LANES = 16

DH = D // NC
EDGES_PER_TILE = N_EDGES // NS
CHUNK = 80
N_CHUNKS = EDGES_PER_TILE // CHUNK
N_CHUNKS_HALF = N_CHUNKS // 2
ROWS_PER_TILE = 632
N_PAD = NS * ROWS_PER_TILE
CNT_W = LANES

_mesh = plsc.VectorSubcoreMesh(core_axis_name="c", subcore_axis_name="s")


@functools.partial(
    pl.kernel,
    out_type=(
        jax.ShapeDtypeStruct((NC, N_PAD, DH), jnp.float32),
        jax.ShapeDtypeStruct((NC, N_PAD, CNT_W), jnp.float32),
    ),
    mesh=_mesh,
    compiler_params=pltpu.CompilerParams(use_tc_tiling_on_sc=False),
    scratch_types=(
        pltpu.VMEM_SHARED((N_PAD, DH), jnp.float32),
        pltpu.VMEM_SHARED((N_PAD, CNT_W), jnp.float32),
        pltpu.VMEM((ROWS_PER_TILE, DH), jnp.float32),
        pltpu.VMEM((ROWS_PER_TILE, CNT_W), jnp.float32),
        pltpu.VMEM((CHUNK, CNT_W), jnp.float32),
        pltpu.VMEM((CHUNK,), jnp.int32),
        pltpu.VMEM((CHUNK,), jnp.int32),
        pltpu.VMEM((CHUNK, DH), jnp.float32),
        pltpu.SemaphoreType.DMA,
    ),
)
def _sc_aggregate(xt_hbm, src_hbm, dst_hbm, acc_out, cnt_out,
                  acc_sh, cnt_sh, zrow, zcnt, ones_b, idx_s, idx_d, rows, sem):
    c = lax.axis_index("c")
    s = lax.axis_index("s")

    zero16 = jnp.zeros((LANES,), jnp.float32)
    one16 = jnp.ones((LANES,), jnp.float32)

    def fill_zero(i, carry):
        for j in range(DH // LANES):
            zrow[i, pl.ds(j * LANES, LANES)] = zero16
        zcnt[i, :] = zero16
        return carry

    lax.fori_loop(0, ROWS_PER_TILE, fill_zero, 0)

    def fill_one(i, carry):
        ones_b[i, :] = one16
        return carry

    lax.fori_loop(0, CHUNK, fill_one, 0)

    row0 = pl.multiple_of(s * ROWS_PER_TILE, 8)
    pltpu.sync_copy(zrow, acc_sh.at[pl.ds(row0, ROWS_PER_TILE)])
    pltpu.sync_copy(zcnt, cnt_sh.at[pl.ds(row0, ROWS_PER_TILE)])
    plsc.subcore_barrier()

    ebase = s * EDGES_PER_TILE
    x_half = xt_hbm.at[c]

    def step(i, carry):
        off = pl.multiple_of(ebase + i * CHUNK, 8)
        pltpu.sync_copy(src_hbm.at[pl.ds(off, CHUNK)], idx_s)
        pltpu.sync_copy(dst_hbm.at[pl.ds(off, CHUNK)], idx_d)
        pltpu.async_copy(x_half.at[idx_s], rows, sem).wait()
        pltpu.sync_copy(rows, acc_sh.at[idx_d], add=True)

        do_cnt = (i < N_CHUNKS_HALF) == (c == 0)

        @pl.when(do_cnt)
        def _():
            pltpu.sync_copy(ones_b, cnt_sh.at[idx_d], add=True)

        return carry

    lax.fori_loop(0, N_CHUNKS, step, 0)
    plsc.subcore_barrier()

    pltpu.sync_copy(acc_sh.at[pl.ds(row0, ROWS_PER_TILE)], zrow)
    pltpu.sync_copy(zrow, acc_out.at[c, pl.ds(row0, ROWS_PER_TILE)])
    pltpu.sync_copy(cnt_sh.at[pl.ds(row0, ROWS_PER_TILE)], zcnt)
    pltpu.sync_copy(zcnt, cnt_out.at[c, pl.ds(row0, ROWS_PER_TILE)])


_BR = 400


def _tc_body(p_ref, cnt_ref, x_ref, wl_ref, wr_ref, b_ref, o_ref):
    p = jnp.concatenate([p_ref[0], p_ref[1]], axis=-1)
    cnt = cnt_ref[:, 0] + cnt_ref[:, 1]
    scale = 1.0 / jnp.maximum(cnt, 1.0)
    aggr = p * scale[:, None]
    dn = (((1,), (1,)), ((), ()))
    o_ref[...] = (
        lax.dot_general(aggr, wl_ref[...], dn, preferred_element_type=jnp.float32)
        + lax.dot_general(x_ref[...], wr_ref[...], dn, preferred_element_type=jnp.float32)
        + b_ref[...]
    )


def _tc_finish(p, cnt, x, W_l, W_r, b):
    grid = (N_NODES // _BR,)
    return pl.pallas_call(
        _tc_body,
        grid=grid,
        in_specs=[
            pl.BlockSpec((NC, _BR, DH), lambda i: (0, i, 0)),
            pl.BlockSpec((_BR, NC), lambda i: (i, 0)),
            pl.BlockSpec((_BR, D), lambda i: (i, 0)),
            pl.BlockSpec((D, D), lambda i: (0, 0)),
            pl.BlockSpec((D, D), lambda i: (0, 0)),
            pl.BlockSpec((1, D), lambda i: (0, 0)),
        ],
        out_specs=pl.BlockSpec((_BR, D), lambda i: (i, 0)),
        out_shape=jax.ShapeDtypeStruct((N_NODES, D), jnp.float32),
    )(p, cnt, x, W_l, W_r, b)


def kernel(x, edge_index, W_l, W_r, b):
    src = edge_index[0]
    dst = edge_index[1]
    xt = x.reshape(N_NODES, NC, DH).transpose(1, 0, 2)
    p, cnt_w = _sc_aggregate(xt, src, dst)
    cnt = cnt_w[:, :, 0].T
    return _tc_finish(p, cnt, x, W_l, W_r, b.reshape(1, D))


# speedup vs baseline: 7.4665x; 2.0994x over previous
import functools

import jax
import jax.numpy as jnp
from jax import lax
from jax.experimental import pallas as pl
from jax.experimental.pallas import tpu as pltpu
from jax.experimental.pallas import tpu_sc as plsc

N_NODES = 10000
N_EDGES = 320000
D = 128

NC = 2
NS = 16
LANES = 16

DH = D // NC
EDGES_PER_TILE = N_EDGES // NS
CHUNK = 125
N_CHUNKS = EDGES_PER_TILE // CHUNK
N_PAIRS = N_CHUNKS // 2
N_CHUNKS_HALF = N_CHUNKS // 2
ROWS_PER_TILE = 632
N_PAD = NS * ROWS_PER_TILE
CNT_W = LANES

_mesh = plsc.VectorSubcoreMesh(core_axis_name="c", subcore_axis_name="s")


@functools.partial(
    pl.kernel,
    out_type=(
        jax.ShapeDtypeStruct((NC, N_PAD, DH), jnp.float32),
        jax.ShapeDtypeStruct((NC, N_PAD, CNT_W), jnp.float32),
    ),
    mesh=_mesh,
    compiler_params=pltpu.CompilerParams(use_tc_tiling_on_sc=False),
    scratch_types=(
        pltpu.VMEM_SHARED((N_PAD, DH), jnp.float32),
        pltpu.VMEM_SHARED((N_PAD, CNT_W), jnp.float32),
        pltpu.VMEM((ROWS_PER_TILE, DH), jnp.float32),
        pltpu.VMEM((ROWS_PER_TILE, CNT_W), jnp.float32),
        pltpu.VMEM((CHUNK, CNT_W), jnp.float32),
        pltpu.VMEM((2, CHUNK), jnp.int32),
        pltpu.VMEM((2, CHUNK), jnp.int32),
        pltpu.VMEM((CHUNK, DH), jnp.float32),
        pltpu.VMEM((CHUNK, DH), jnp.float32),
        pltpu.SemaphoreType.DMA,
        pltpu.SemaphoreType.DMA,
        pltpu.SemaphoreType.DMA,
        pltpu.SemaphoreType.DMA,
    ),
)
def _sc_aggregate(xt_hbm, eidx_hbm, acc_out, cnt_out,
                  acc_sh, cnt_sh, zrow, zcnt, ones_b,
                  idx_a, idx_b, rows_a, rows_b, sem_ia, sem_ib, sem_a, sem_b):
    c = lax.axis_index("c")
    s = lax.axis_index("s")

    zero16 = jnp.zeros((LANES,), jnp.float32)
    one16 = jnp.ones((LANES,), jnp.float32)

    def fill_zero(i, carry):
        for j in range(DH // LANES):
            zrow[i, pl.ds(j * LANES, LANES)] = zero16
        zcnt[i, :] = zero16
        return carry

    lax.fori_loop(0, ROWS_PER_TILE, fill_zero, 0)

    def fill_one(i, carry):
        ones_b[i, :] = one16
        return carry

    lax.fori_loop(0, CHUNK, fill_one, 0)

    my_eidx = eidx_hbm.at[s]

    def idx_start(chunk, buf, sem):
        pltpu.async_copy(my_eidx.at[chunk], buf, sem)

    def idx_wait(chunk, buf, sem):
        pltpu.make_async_copy(my_eidx.at[chunk], buf, sem).wait()

    row0 = pl.multiple_of(s * ROWS_PER_TILE, 8)
    idx_start(0, idx_a, sem_ia)
    idx_start(1, idx_b, sem_ib)
    pltpu.sync_copy(zrow, acc_sh.at[pl.ds(row0, ROWS_PER_TILE)])
    pltpu.sync_copy(zcnt, cnt_sh.at[pl.ds(row0, ROWS_PER_TILE)])
    plsc.subcore_barrier()

    x_half = xt_hbm.at[c]

    def gather_start(ibuf, buf, sem):
        pltpu.async_copy(x_half.at[ibuf.at[0]], buf, sem)

    def gather_wait(ibuf, buf, sem):
        pltpu.make_async_copy(x_half.at[ibuf.at[0]], buf, sem).wait()

    def scatter(chunk, ibuf, buf):
        pltpu.sync_copy(buf, acc_sh.at[ibuf.at[1]], add=True)
        do_cnt = (chunk < N_CHUNKS_HALF) == (c == 0)

        @pl.when(do_cnt)
        def _():
            pltpu.sync_copy(ones_b, cnt_sh.at[ibuf.at[1]], add=True)

    idx_wait(0, idx_a, sem_ia)
    gather_start(idx_a, rows_a, sem_a)

    def step(k, carry):
        i = 2 * k
        idx_wait(i + 1, idx_b, sem_ib)
        gather_start(idx_b, rows_b, sem_b)

        gather_wait(idx_a, rows_a, sem_a)
        scatter(i, idx_a, rows_a)

        @pl.when(k < N_PAIRS - 1)
        def _():
            idx_start(i + 2, idx_a, sem_ia)
            idx_wait(i + 2, idx_a, sem_ia)
            gather_start(idx_a, rows_a, sem_a)

        gather_wait(idx_b, rows_b, sem_b)
        scatter(i + 1, idx_b, rows_b)

        @pl.when(k < N_PAIRS - 1)
        def _():
            idx_start(i + 3, idx_b, sem_ib)

        return carry

    lax.fori_loop(0, N_PAIRS, step, 0)
    plsc.subcore_barrier()

    pltpu.sync_copy(acc_sh.at[pl.ds(row0, ROWS_PER_TILE)], zrow)
    pltpu.sync_copy(zrow, acc_out.at[c, pl.ds(row0, ROWS_PER_TILE)])
    pltpu.sync_copy(cnt_sh.at[pl.ds(row0, ROWS_PER_TILE)], zcnt)
    pltpu.sync_copy(zcnt, cnt_out.at[c, pl.ds(row0, ROWS_PER_TILE)])


_BR = 400


def _tc_body(p_ref, cnt_ref, x_ref, wl_ref, wr_ref, b_ref, o_ref):
    p = jnp.concatenate([p_ref[0], p_ref[1]], axis=-1)
    cnt = cnt_ref[:, 0] + cnt_ref[:, 1]
    scale = 1.0 / jnp.maximum(cnt, 1.0)
    aggr = p * scale[:, None]
    dn = (((1,), (1,)), ((), ()))
    o_ref[...] = (
        lax.dot_general(aggr, wl_ref[...], dn, preferred_element_type=jnp.float32)
        + lax.dot_general(x_ref[...], wr_ref[...], dn, preferred_element_type=jnp.float32)
        + b_ref[...]
    )


def _tc_finish(p, cnt, x, W_l, W_r, b):
    grid = (N_NODES // _BR,)
    return pl.pallas_call(
        _tc_body,
        grid=grid,
        in_specs=[
            pl.BlockSpec((NC, _BR, DH), lambda i: (0, i, 0)),
            pl.BlockSpec((_BR, NC), lambda i: (i, 0)),
            pl.BlockSpec((_BR, D), lambda i: (i, 0)),
            pl.BlockSpec((D, D), lambda i: (0, 0)),
            pl.BlockSpec((D, D), lambda i: (0, 0)),
            pl.BlockSpec((1, D), lambda i: (0, 0)),
        ],
        out_specs=pl.BlockSpec((_BR, D), lambda i: (i, 0)),
        out_shape=jax.ShapeDtypeStruct((N_NODES, D), jnp.float32),
    )(p, cnt, x, W_l, W_r, b)


def kernel(x, edge_index, W_l, W_r, b):
    xt = x.reshape(N_NODES, NC, DH).transpose(1, 0, 2)
    eidx = edge_index.reshape(2, NS, N_CHUNKS, CHUNK).transpose(1, 2, 0, 3)
    p, cnt_w = _sc_aggregate(xt, eidx)
    cnt = cnt_w[:, :, 0].T
    return _tc_finish(p, cnt, x, W_l, W_r, b.reshape(1, D))


# speedup vs baseline: 9.2131x; 1.2339x over previous
import functools

import jax
import jax.numpy as jnp
from jax import lax
from jax.experimental import pallas as pl
from jax.experimental.pallas import tpu as pltpu
from jax.experimental.pallas import tpu_sc as plsc

N_NODES = 10000
N_EDGES = 320000
D = 128

NC = 2
NS = 16
LANES = 16

DH = D // NC
EDGES_PER_TILE = N_EDGES // NS
CHUNK = 125
N_CHUNKS = EDGES_PER_TILE // CHUNK
N_PAIRS = N_CHUNKS // 2
N_CHUNKS_HALF = N_CHUNKS // 2
ROWS_PER_TILE = 632
N_PAD = NS * ROWS_PER_TILE
CNT_W = LANES

_mesh = plsc.VectorSubcoreMesh(core_axis_name="c", subcore_axis_name="s")


@functools.partial(
    pl.kernel,
    out_type=(
        jax.ShapeDtypeStruct((NC, N_PAD, DH), jnp.float32),
        jax.ShapeDtypeStruct((NC, N_PAD, CNT_W), jnp.float32),
    ),
    mesh=_mesh,
    compiler_params=pltpu.CompilerParams(use_tc_tiling_on_sc=False),
    scratch_types=(
        pltpu.VMEM_SHARED((N_PAD, DH), jnp.float32),
        pltpu.VMEM_SHARED((N_PAD, CNT_W), jnp.float32),
        pltpu.VMEM((ROWS_PER_TILE, DH), jnp.float32),
        pltpu.VMEM((ROWS_PER_TILE, CNT_W), jnp.float32),
        pltpu.VMEM((CHUNK, CNT_W), jnp.float32),
        pltpu.VMEM((2, CHUNK), jnp.int32),
        pltpu.VMEM((2, CHUNK), jnp.int32),
        pltpu.VMEM((CHUNK, DH), jnp.float32),
        pltpu.VMEM((CHUNK, DH), jnp.float32),
        pltpu.SemaphoreType.DMA,
        pltpu.SemaphoreType.DMA,
        pltpu.SemaphoreType.DMA,
        pltpu.SemaphoreType.DMA,
    ),
)
def _sc_aggregate(xt_hbm, eidx_hbm, acc_out, cnt_out,
                  acc_sh, cnt_sh, zrow, zcnt, ones_b,
                  idx_a, idx_b, rows_a, rows_b, sem_ia, sem_ib, sem_a, sem_b):
    c = lax.axis_index("c")
    s = lax.axis_index("s")

    zero16 = jnp.zeros((LANES,), jnp.float32)
    one16 = jnp.ones((LANES,), jnp.float32)

    def fill_zero(i, carry):
        for j in range(DH // LANES):
            zrow[i, pl.ds(j * LANES, LANES)] = zero16
        zcnt[i, :] = zero16
        return carry

    lax.fori_loop(0, ROWS_PER_TILE, fill_zero, 0)

    def fill_one(i, carry):
        ones_b[i, :] = one16
        return carry

    lax.fori_loop(0, CHUNK, fill_one, 0)

    my_src = eidx_hbm.at[0, s]
    my_dst = eidx_hbm.at[1, s]

    def idx_start(chunk, buf, sem):
        pltpu.async_copy(my_src.at[chunk], buf.at[0], sem)
        pltpu.async_copy(my_dst.at[chunk], buf.at[1], sem)

    def idx_wait(chunk, buf, sem):
        pltpu.make_async_copy(my_src.at[chunk], buf.at[0], sem).wait()
        pltpu.make_async_copy(my_dst.at[chunk], buf.at[1], sem).wait()

    row0 = pl.multiple_of(s * ROWS_PER_TILE, 8)
    idx_start(0, idx_a, sem_ia)
    idx_start(1, idx_b, sem_ib)
    pltpu.sync_copy(zrow, acc_sh.at[pl.ds(row0, ROWS_PER_TILE)])
    pltpu.sync_copy(zcnt, cnt_sh.at[pl.ds(row0, ROWS_PER_TILE)])
    plsc.subcore_barrier()

    x_half = xt_hbm.at[c]

    def gather_start(ibuf, buf, sem):
        pltpu.async_copy(x_half.at[ibuf.at[0]], buf, sem)

    def gather_wait(ibuf, buf, sem):
        pltpu.make_async_copy(x_half.at[ibuf.at[0]], buf, sem).wait()

    def scatter(chunk, ibuf, buf):
        pltpu.sync_copy(buf, acc_sh.at[ibuf.at[1]], add=True)
        do_cnt = (chunk < N_CHUNKS_HALF) == (c == 0)

        @pl.when(do_cnt)
        def _():
            pltpu.sync_copy(ones_b, cnt_sh.at[ibuf.at[1]], add=True)

    idx_wait(0, idx_a, sem_ia)
    gather_start(idx_a, rows_a, sem_a)

    def step(k, carry):
        i = 2 * k
        idx_wait(i + 1, idx_b, sem_ib)
        gather_start(idx_b, rows_b, sem_b)

        gather_wait(idx_a, rows_a, sem_a)
        scatter(i, idx_a, rows_a)

        @pl.when(k < N_PAIRS - 1)
        def _():
            idx_start(i + 2, idx_a, sem_ia)
            idx_wait(i + 2, idx_a, sem_ia)
            gather_start(idx_a, rows_a, sem_a)

        gather_wait(idx_b, rows_b, sem_b)
        scatter(i + 1, idx_b, rows_b)

        @pl.when(k < N_PAIRS - 1)
        def _():
            idx_start(i + 3, idx_b, sem_ib)

        return carry

    lax.fori_loop(0, N_PAIRS, step, 0)
    plsc.subcore_barrier()

    pltpu.sync_copy(acc_sh.at[pl.ds(row0, ROWS_PER_TILE)], zrow)
    pltpu.sync_copy(zrow, acc_out.at[c, pl.ds(row0, ROWS_PER_TILE)])
    pltpu.sync_copy(cnt_sh.at[pl.ds(row0, ROWS_PER_TILE)], zcnt)
    pltpu.sync_copy(zcnt, cnt_out.at[c, pl.ds(row0, ROWS_PER_TILE)])


_BR = 400


def _tc_body(p_ref, cnt_ref, x_ref, wl_ref, wr_ref, b_ref, o_ref):
    p = jnp.concatenate([p_ref[0], p_ref[1]], axis=-1)
    cnt = cnt_ref[0, :, :1] + cnt_ref[1, :, :1]
    scale = 1.0 / jnp.maximum(cnt, 1.0)
    aggr = p * scale
    dn = (((1,), (1,)), ((), ()))
    o_ref[...] = (
        lax.dot_general(aggr, wl_ref[...], dn, preferred_element_type=jnp.float32)
        + lax.dot_general(x_ref[...], wr_ref[...], dn, preferred_element_type=jnp.float32)
        + b_ref[...]
    )


def _tc_finish(p, cnt, x, W_l, W_r, b):
    grid = (N_NODES // _BR,)
    return pl.pallas_call(
        _tc_body,
        grid=grid,
        in_specs=[
            pl.BlockSpec((NC, _BR, DH), lambda i: (0, i, 0)),
            pl.BlockSpec((NC, _BR, CNT_W), lambda i: (0, i, 0)),
            pl.BlockSpec((_BR, D), lambda i: (i, 0)),
            pl.BlockSpec((D, D), lambda i: (0, 0)),
            pl.BlockSpec((D, D), lambda i: (0, 0)),
            pl.BlockSpec((1, D), lambda i: (0, 0)),
        ],
        out_specs=pl.BlockSpec((_BR, D), lambda i: (i, 0)),
        out_shape=jax.ShapeDtypeStruct((N_NODES, D), jnp.float32),
    )(p, cnt, x, W_l, W_r, b)


def kernel(x, edge_index, W_l, W_r, b):
    xt = x.reshape(N_NODES, NC, DH).transpose(1, 0, 2)
    eidx = edge_index.reshape(2, NS, N_CHUNKS, CHUNK)
    p, cnt_w = _sc_aggregate(xt, eidx)
    return _tc_finish(p, cnt_w, x, W_l, W_r, b.reshape(1, D))


# speedup vs baseline: 9.9675x; 1.0819x over previous
import functools

import jax
import jax.numpy as jnp
from jax import lax
from jax.experimental import pallas as pl
from jax.experimental.pallas import tpu as pltpu
from jax.experimental.pallas import tpu_sc as plsc

N_NODES = 10000
N_EDGES = 320000
D = 128

NC = 2
NS = 16
LANES = 16

DH = D // NC
EDGES_PER_TILE = N_EDGES // NS
CHUNK = 125
N_CHUNKS = EDGES_PER_TILE // CHUNK
N_PAIRS = N_CHUNKS // 2
N_CHUNKS_HALF = N_CHUNKS // 2
ROWS_PER_TILE = 632
N_PAD = NS * ROWS_PER_TILE
CNT_W = LANES

_mesh = plsc.VectorSubcoreMesh(core_axis_name="c", subcore_axis_name="s")


@functools.partial(
    pl.kernel,
    out_type=(
        jax.ShapeDtypeStruct((NC, N_PAD, DH), jnp.float32),
        jax.ShapeDtypeStruct((NC, N_PAD, CNT_W), jnp.float32),
    ),
    mesh=_mesh,
    compiler_params=pltpu.CompilerParams(use_tc_tiling_on_sc=False),
    scratch_types=(
        pltpu.VMEM_SHARED((N_PAD, DH), jnp.float32),
        pltpu.VMEM_SHARED((N_PAD, CNT_W), jnp.float32),
        pltpu.VMEM((ROWS_PER_TILE, DH), jnp.float32),
        pltpu.VMEM((ROWS_PER_TILE, CNT_W), jnp.float32),
        pltpu.VMEM((CHUNK, CNT_W), jnp.float32),
        pltpu.VMEM((2, CHUNK), jnp.int32),
        pltpu.VMEM((2, CHUNK), jnp.int32),
        pltpu.VMEM((CHUNK, DH), jnp.float32),
        pltpu.VMEM((CHUNK, DH), jnp.float32),
        pltpu.SemaphoreType.DMA,
        pltpu.SemaphoreType.DMA,
        pltpu.SemaphoreType.DMA,
        pltpu.SemaphoreType.DMA,
    ),
)
def _sc_aggregate(xt_hbm, eidx_hbm, acc_out, cnt_out,
                  acc_sh, cnt_sh, zrow, zcnt, ones_b,
                  idx_a, idx_b, rows_a, rows_b, sem_ia, sem_ib, sem_a, sem_b):
    c = lax.axis_index("c")
    s = lax.axis_index("s")

    zero16 = jnp.zeros((LANES,), jnp.float32)
    one16 = jnp.ones((LANES,), jnp.float32)

    def fill_zero(i, carry):
        for j in range(DH // LANES):
            zrow[i, pl.ds(j * LANES, LANES)] = zero16
        zcnt[i, :] = zero16
        return carry

    lax.fori_loop(0, ROWS_PER_TILE, fill_zero, 0)

    def fill_one(i, carry):
        ones_b[i, :] = one16
        return carry

    lax.fori_loop(0, CHUNK, fill_one, 0)

    my_src = eidx_hbm.at[0, s]
    my_dst = eidx_hbm.at[1, s]

    def idx_start(chunk, buf, sem):
        pltpu.async_copy(my_src.at[chunk], buf.at[0], sem)
        pltpu.async_copy(my_dst.at[chunk], buf.at[1], sem)

    def idx_wait(chunk, buf, sem):
        pltpu.make_async_copy(my_src.at[chunk], buf.at[0], sem).wait()
        pltpu.make_async_copy(my_dst.at[chunk], buf.at[1], sem).wait()

    row0 = pl.multiple_of(s * ROWS_PER_TILE, 8)
    idx_start(0, idx_a, sem_ia)
    idx_start(1, idx_b, sem_ib)
    pltpu.sync_copy(zrow, acc_sh.at[pl.ds(row0, ROWS_PER_TILE)])
    pltpu.sync_copy(zcnt, cnt_sh.at[pl.ds(row0, ROWS_PER_TILE)])
    plsc.subcore_barrier()

    x_half = xt_hbm.at[pl.ds(c, 2 * N_NODES - 1)]

    def gather_start(ibuf, buf, sem):
        pltpu.async_copy(x_half.at[ibuf.at[0]], buf, sem)

    def gather_wait(ibuf, buf, sem):
        pltpu.make_async_copy(x_half.at[ibuf.at[0]], buf, sem).wait()

    def scatter(chunk, ibuf, buf):
        pltpu.sync_copy(buf, acc_sh.at[ibuf.at[1]], add=True)
        do_cnt = (chunk < N_CHUNKS_HALF) == (c == 0)

        @pl.when(do_cnt)
        def _():
            pltpu.sync_copy(ones_b, cnt_sh.at[ibuf.at[1]], add=True)

    idx_wait(0, idx_a, sem_ia)
    gather_start(idx_a, rows_a, sem_a)

    def step(k, carry):
        i = 2 * k
        idx_wait(i + 1, idx_b, sem_ib)
        gather_start(idx_b, rows_b, sem_b)

        gather_wait(idx_a, rows_a, sem_a)
        scatter(i, idx_a, rows_a)

        @pl.when(k < N_PAIRS - 1)
        def _():
            idx_start(i + 2, idx_a, sem_ia)
            idx_wait(i + 2, idx_a, sem_ia)
            gather_start(idx_a, rows_a, sem_a)

        gather_wait(idx_b, rows_b, sem_b)
        scatter(i + 1, idx_b, rows_b)

        @pl.when(k < N_PAIRS - 1)
        def _():
            idx_start(i + 3, idx_b, sem_ib)

        return carry

    lax.fori_loop(0, N_PAIRS, step, 0)
    plsc.subcore_barrier()

    pltpu.sync_copy(acc_sh.at[pl.ds(row0, ROWS_PER_TILE)], zrow)
    pltpu.sync_copy(zrow, acc_out.at[c, pl.ds(row0, ROWS_PER_TILE)])
    pltpu.sync_copy(cnt_sh.at[pl.ds(row0, ROWS_PER_TILE)], zcnt)
    pltpu.sync_copy(zcnt, cnt_out.at[c, pl.ds(row0, ROWS_PER_TILE)])


_BR = 400


def _tc_self_body(x_ref, wr_ref, b_ref, y_ref):
    dn = (((1,), (1,)), ((), ()))
    y_ref[...] = (
        lax.dot_general(x_ref[...], wr_ref[...], dn, preferred_element_type=jnp.float32)
        + b_ref[...]
    )


def _tc_self(x, W_r, b):
    return pl.pallas_call(
        _tc_self_body,
        grid=(N_NODES // _BR,),
        in_specs=[
            pl.BlockSpec((_BR, D), lambda i: (i, 0)),
            pl.BlockSpec((D, D), lambda i: (0, 0)),
            pl.BlockSpec((1, D), lambda i: (0, 0)),
        ],
        out_specs=pl.BlockSpec((_BR, D), lambda i: (i, 0)),
        out_shape=jax.ShapeDtypeStruct((N_NODES, D), jnp.float32),
    )(x, W_r, b)


def _tc_body(p_ref, cnt_ref, y_ref, wl_ref, o_ref):
    p = jnp.concatenate([p_ref[0], p_ref[1]], axis=-1)
    cnt = cnt_ref[0, :, :1] + cnt_ref[1, :, :1]
    scale = 1.0 / jnp.maximum(cnt, 1.0)
    aggr = p * scale
    dn = (((1,), (1,)), ((), ()))
    o_ref[...] = (
        lax.dot_general(aggr, wl_ref[...], dn, preferred_element_type=jnp.float32)
        + y_ref[...]
    )


def _tc_finish(p, cnt, y, W_l):
    grid = (N_NODES // _BR,)
    return pl.pallas_call(
        _tc_body,
        grid=grid,
        in_specs=[
            pl.BlockSpec((NC, _BR, DH), lambda i: (0, i, 0)),
            pl.BlockSpec((NC, _BR, CNT_W), lambda i: (0, i, 0)),
            pl.BlockSpec((_BR, D), lambda i: (i, 0)),
            pl.BlockSpec((D, D), lambda i: (0, 0)),
        ],
        out_specs=pl.BlockSpec((_BR, D), lambda i: (i, 0)),
        out_shape=jax.ShapeDtypeStruct((N_NODES, D), jnp.float32),
    )(p, cnt, y, W_l)


def kernel(x, edge_index, W_l, W_r, b):
    x2 = x.reshape(NC * N_NODES, DH)
    eidx = (edge_index * jnp.array([[2], [1]], dtype=jnp.int32)).reshape(
        2, NS, N_CHUNKS, CHUNK)
    y = _tc_self(x, W_r, b.reshape(1, D))
    p, cnt_w = _sc_aggregate(x2, eidx)
    return _tc_finish(p, cnt_w, y, W_l)


# speedup vs baseline: 10.2084x; 1.0242x over previous
import functools

import jax
import jax.numpy as jnp
from jax import lax
from jax.experimental import pallas as pl
from jax.experimental.pallas import tpu as pltpu
from jax.experimental.pallas import tpu_sc as plsc

N_NODES = 10000
N_EDGES = 320000
D = 128

NC = 2
NS = 16
LANES = 16

DH = D // NC
EDGES_PER_TILE = N_EDGES // NS
CHUNK = 125
N_CHUNKS = EDGES_PER_TILE // CHUNK
DEPTH = 4
N_GROUPS = N_CHUNKS // DEPTH
N_CHUNKS_HALF = N_CHUNKS // 2
ROWS_PER_TILE = 632
N_PAD = NS * ROWS_PER_TILE
Q_STAGE = 4
QROWS = ROWS_PER_TILE // Q_STAGE
CNT_W = LANES

_mesh = plsc.VectorSubcoreMesh(core_axis_name="c", subcore_axis_name="s")


@functools.partial(
    pl.kernel,
    out_type=(
        jax.ShapeDtypeStruct((NC, N_PAD, DH), jnp.float32),
        jax.ShapeDtypeStruct((NC, N_PAD, CNT_W), jnp.float32),
    ),
    mesh=_mesh,
    compiler_params=pltpu.CompilerParams(use_tc_tiling_on_sc=False),
    scratch_types=(
        pltpu.VMEM_SHARED((N_PAD, DH), jnp.float32),
        pltpu.VMEM_SHARED((N_PAD, CNT_W), jnp.float32),
        pltpu.VMEM((QROWS, DH), jnp.float32),
        pltpu.VMEM((QROWS, CNT_W), jnp.float32),
        pltpu.VMEM((CHUNK, CNT_W), jnp.float32),
        [pltpu.VMEM((2, CHUNK), jnp.int32) for _ in range(DEPTH)],
        [pltpu.VMEM((CHUNK, DH), jnp.float32) for _ in range(DEPTH)],
        [pltpu.SemaphoreType.DMA for _ in range(DEPTH)],
        [pltpu.SemaphoreType.DMA for _ in range(DEPTH)],
    ),
)
def _sc_aggregate(xt_hbm, eidx_hbm, acc_out, cnt_out,
                  acc_sh, cnt_sh, zrow, zcnt, ones_b,
                  idx_bufs, row_bufs, idx_sems, row_sems):
    c = lax.axis_index("c")
    s = lax.axis_index("s")

    zero16 = jnp.zeros((LANES,), jnp.float32)
    one16 = jnp.ones((LANES,), jnp.float32)

    def fill_zero(i, carry):
        for j in range(DH // LANES):
            zrow[i, pl.ds(j * LANES, LANES)] = zero16
        zcnt[i, :] = zero16
        return carry

    lax.fori_loop(0, QROWS, fill_zero, 0)

    def fill_one(i, carry):
        ones_b[i, :] = one16
        return carry

    lax.fori_loop(0, CHUNK, fill_one, 0)

    my_src = eidx_hbm.at[0, s]
    my_dst = eidx_hbm.at[1, s]

    def idx_start(chunk, buf, sem):
        pltpu.async_copy(my_src.at[chunk], buf.at[0], sem)
        pltpu.async_copy(my_dst.at[chunk], buf.at[1], sem)

    def idx_wait(chunk, buf, sem):
        pltpu.make_async_copy(my_src.at[chunk], buf.at[0], sem).wait()
        pltpu.make_async_copy(my_dst.at[chunk], buf.at[1], sem).wait()

    row0 = pl.multiple_of(s * ROWS_PER_TILE, 8)
    for u in range(DEPTH):
        idx_start(u, idx_bufs[u], idx_sems[u])
    for q in range(Q_STAGE):
        pltpu.sync_copy(zrow, acc_sh.at[pl.ds(row0 + q * QROWS, QROWS)])
        pltpu.sync_copy(zcnt, cnt_sh.at[pl.ds(row0 + q * QROWS, QROWS)])
    plsc.subcore_barrier()

    x_half = xt_hbm.at[pl.ds(c, 2 * N_NODES - 1)]

    def gather_start(ibuf, buf, sem):
        pltpu.async_copy(x_half.at[ibuf.at[0]], buf, sem)

    def gather_wait(ibuf, buf, sem):
        pltpu.make_async_copy(x_half.at[ibuf.at[0]], buf, sem).wait()

    def scatter(chunk, ibuf, buf):
        pltpu.sync_copy(buf, acc_sh.at[ibuf.at[1]], add=True)
        do_cnt = (chunk < N_CHUNKS_HALF) == (c == 0)

        @pl.when(do_cnt)
        def _():
            pltpu.sync_copy(ones_b, cnt_sh.at[ibuf.at[1]], add=True)

    for u in range(DEPTH):
        idx_wait(u, idx_bufs[u], idx_sems[u])
        gather_start(idx_bufs[u], row_bufs[u], row_sems[u])

    def step(k, carry):
        i = DEPTH * k
        for u in range(DEPTH):
            gather_wait(idx_bufs[u], row_bufs[u], row_sems[u])
            scatter(i + u, idx_bufs[u], row_bufs[u])

            @pl.when(k < N_GROUPS - 1)
            def _():
                idx_start(i + u + DEPTH, idx_bufs[u], idx_sems[u])
                idx_wait(i + u + DEPTH, idx_bufs[u], idx_sems[u])
                gather_start(idx_bufs[u], row_bufs[u], row_sems[u])

        return carry

    lax.fori_loop(0, N_GROUPS, step, 0)
    plsc.subcore_barrier()

    for q in range(Q_STAGE):
        r = row0 + q * QROWS
        pltpu.sync_copy(acc_sh.at[pl.ds(r, QROWS)], zrow)
        pltpu.sync_copy(zrow, acc_out.at[c, pl.ds(r, QROWS)])
        pltpu.sync_copy(cnt_sh.at[pl.ds(r, QROWS)], zcnt)
        pltpu.sync_copy(zcnt, cnt_out.at[c, pl.ds(r, QROWS)])


_BR = 400


def _tc_self_body(x_ref, wr_ref, b_ref, y_ref):
    dn = (((1,), (1,)), ((), ()))
    y_ref[...] = (
        lax.dot_general(x_ref[...], wr_ref[...], dn, preferred_element_type=jnp.float32)
        + b_ref[...]
    )


def _tc_self(x, W_r, b):
    return pl.pallas_call(
        _tc_self_body,
        grid=(N_NODES // _BR,),
        in_specs=[
            pl.BlockSpec((_BR, D), lambda i: (i, 0)),
            pl.BlockSpec((D, D), lambda i: (0, 0)),
            pl.BlockSpec((1, D), lambda i: (0, 0)),
        ],
        out_specs=pl.BlockSpec((_BR, D), lambda i: (i, 0)),
        out_shape=jax.ShapeDtypeStruct((N_NODES, D), jnp.float32),
    )(x, W_r, b)


def _tc_body(p_ref, cnt_ref, y_ref, wl_ref, o_ref):
    p = jnp.concatenate([p_ref[0], p_ref[1]], axis=-1)
    cnt = cnt_ref[0, :, :1] + cnt_ref[1, :, :1]
    scale = 1.0 / jnp.maximum(cnt, 1.0)
    aggr = p * scale
    dn = (((1,), (1,)), ((), ()))
    o_ref[...] = (
        lax.dot_general(aggr, wl_ref[...], dn, preferred_element_type=jnp.float32)
        + y_ref[...]
    )


def _tc_finish(p, cnt, y, W_l):
    grid = (N_NODES // _BR,)
    return pl.pallas_call(
        _tc_body,
        grid=grid,
        in_specs=[
            pl.BlockSpec((NC, _BR, DH), lambda i: (0, i, 0)),
            pl.BlockSpec((NC, _BR, CNT_W), lambda i: (0, i, 0)),
            pl.BlockSpec((_BR, D), lambda i: (i, 0)),
            pl.BlockSpec((D, D), lambda i: (0, 0)),
        ],
        out_specs=pl.BlockSpec((_BR, D), lambda i: (i, 0)),
        out_shape=jax.ShapeDtypeStruct((N_NODES, D), jnp.float32),
    )(p, cnt, y, W_l)


def kernel(x, edge_index, W_l, W_r, b):
    x2 = x.reshape(NC * N_NODES, DH)
    eidx = (edge_index * jnp.array([[2], [1]], dtype=jnp.int32)).reshape(
        2, NS, N_CHUNKS, CHUNK)
    y = _tc_self(x, W_r, b.reshape(1, D))
    p, cnt_w = _sc_aggregate(x2, eidx)
    return _tc_finish(p, cnt_w, y, W_l)


# speedup vs baseline: 11.3566x; 1.1125x over previous
import functools

import jax
import jax.numpy as jnp
from jax import lax
from jax.experimental import pallas as pl
from jax.experimental.pallas import tpu as pltpu
from jax.experimental.pallas import tpu_sc as plsc

N_NODES = 10000
N_EDGES = 320000
D = 128

NC = 2
NS = 16
LANES = 16

DH = D // NC
EDGES_PER_TILE = N_EDGES // NS
CHUNK = 125
N_CHUNKS = EDGES_PER_TILE // CHUNK
DEPTH = 4
N_GROUPS = N_CHUNKS // DEPTH
N_CHUNKS_HALF = N_CHUNKS // 2
ROWS_PER_TILE = 632
N_PAD = NS * ROWS_PER_TILE
Q_STAGE = 4
QROWS = ROWS_PER_TILE // Q_STAGE
CNT_W = LANES

_mesh = plsc.VectorSubcoreMesh(core_axis_name="c", subcore_axis_name="s")


@functools.partial(
    pl.kernel,
    out_type=(
        jax.ShapeDtypeStruct((NC, N_PAD, DH), jnp.float32),
        jax.ShapeDtypeStruct((NC, N_PAD, CNT_W), jnp.float32),
    ),
    mesh=_mesh,
    compiler_params=pltpu.CompilerParams(use_tc_tiling_on_sc=False),
    scratch_types=(
        pltpu.VMEM_SHARED((N_PAD, DH), jnp.float32),
        pltpu.VMEM_SHARED((N_PAD, CNT_W), jnp.float32),
        pltpu.VMEM((QROWS, DH), jnp.float32),
        pltpu.VMEM((QROWS, CNT_W), jnp.float32),
        pltpu.VMEM((CHUNK, CNT_W), jnp.float32),
        [pltpu.VMEM((2, CHUNK), jnp.int32) for _ in range(DEPTH)],
        [pltpu.VMEM((CHUNK, DH), jnp.float32) for _ in range(DEPTH)],
        [pltpu.SemaphoreType.DMA for _ in range(DEPTH)],
        [pltpu.SemaphoreType.DMA for _ in range(DEPTH)],
        [pltpu.SemaphoreType.DMA for _ in range(DEPTH)],
        [pltpu.SemaphoreType.DMA for _ in range(DEPTH)],
    ),
)
def _sc_aggregate(xt_hbm, eidx_hbm, acc_out, cnt_out,
                  acc_sh, cnt_sh, zrow, zcnt, ones_b,
                  idx_bufs, row_bufs, idx_sems, row_sems, scat_sems, cnt_sems):
    c = lax.axis_index("c")
    s = lax.axis_index("s")

    zero16 = jnp.zeros((LANES,), jnp.float32)
    one16 = jnp.ones((LANES,), jnp.float32)

    def fill_zero(i, carry):
        for j in range(DH // LANES):
            zrow[i, pl.ds(j * LANES, LANES)] = zero16
        zcnt[i, :] = zero16
        return carry

    lax.fori_loop(0, QROWS, fill_zero, 0)

    def fill_one(i, carry):
        ones_b[i, :] = one16
        return carry

    lax.fori_loop(0, CHUNK, fill_one, 0)

    my_src = eidx_hbm.at[0, s]
    my_dst = eidx_hbm.at[1, s]

    def idx_start(chunk, buf, sem):
        pltpu.async_copy(my_src.at[chunk], buf.at[0], sem)
        pltpu.async_copy(my_dst.at[chunk], buf.at[1], sem)

    def idx_wait(chunk, buf, sem):
        pltpu.make_async_copy(my_src.at[chunk], buf.at[0], sem).wait()
        pltpu.make_async_copy(my_dst.at[chunk], buf.at[1], sem).wait()

    row0 = pl.multiple_of(s * ROWS_PER_TILE, 8)
    for u in range(DEPTH):
        idx_start(u, idx_bufs[u], idx_sems[u])
    for q in range(Q_STAGE):
        pltpu.sync_copy(zrow, acc_sh.at[pl.ds(row0 + q * QROWS, QROWS)])
        pltpu.sync_copy(zcnt, cnt_sh.at[pl.ds(row0 + q * QROWS, QROWS)])
    plsc.subcore_barrier()

    x_half = xt_hbm.at[pl.ds(c, 2 * N_NODES - 1)]

    def gather_start(ibuf, buf, sem):
        pltpu.async_copy(x_half.at[ibuf.at[0]], buf, sem)

    def gather_wait(ibuf, buf, sem):
        pltpu.make_async_copy(x_half.at[ibuf.at[0]], buf, sem).wait()

    def do_cnt(chunk):
        return (chunk < N_CHUNKS_HALF) == (c == 0)

    def scat_start(chunk, u):
        pltpu.async_copy(row_bufs[u], acc_sh.at[idx_bufs[u].at[1]],
                         scat_sems[u], add=True)

        @pl.when(do_cnt(chunk))
        def _():
            pltpu.async_copy(ones_b, cnt_sh.at[idx_bufs[u].at[1]],
                             cnt_sems[u], add=True)

    def scat_wait(chunk, u):
        pltpu.make_async_copy(row_bufs[u], acc_sh.at[idx_bufs[u].at[1]],
                              scat_sems[u]).wait()

        @pl.when(do_cnt(chunk))
        def _():
            pltpu.make_async_copy(ones_b, cnt_sh.at[idx_bufs[u].at[1]],
                                  cnt_sems[u]).wait()

    for u in range(DEPTH):
        idx_wait(u, idx_bufs[u], idx_sems[u])
        gather_start(idx_bufs[u], row_bufs[u], row_sems[u])

    def step(k, carry):
        i = DEPTH * k
        for u in range(DEPTH):
            gather_wait(idx_bufs[u], row_bufs[u], row_sems[u])
            scat_start(i + u, u)

        for u in range(DEPTH):
            @pl.when(k < N_GROUPS - 1)
            def _():
                scat_wait(i + u, u)
                idx_start(i + u + DEPTH, idx_bufs[u], idx_sems[u])
                idx_wait(i + u + DEPTH, idx_bufs[u], idx_sems[u])
                gather_start(idx_bufs[u], row_bufs[u], row_sems[u])

        return carry

    lax.fori_loop(0, N_GROUPS, step, 0)

    for u in range(DEPTH):
        scat_wait(N_CHUNKS - DEPTH + u, u)
    plsc.subcore_barrier()

    for q in range(Q_STAGE):
        r = row0 + q * QROWS
        pltpu.sync_copy(acc_sh.at[pl.ds(r, QROWS)], zrow)
        pltpu.sync_copy(zrow, acc_out.at[c, pl.ds(r, QROWS)])
        pltpu.sync_copy(cnt_sh.at[pl.ds(r, QROWS)], zcnt)
        pltpu.sync_copy(zcnt, cnt_out.at[c, pl.ds(r, QROWS)])


_BR = 400


def _tc_self_body(x_ref, wr_ref, b_ref, y_ref):
    dn = (((1,), (1,)), ((), ()))
    y_ref[...] = (
        lax.dot_general(x_ref[...], wr_ref[...], dn, preferred_element_type=jnp.float32)
        + b_ref[...]
    )


def _tc_self(x, W_r, b):
    return pl.pallas_call(
        _tc_self_body,
        grid=(N_NODES // _BR,),
        in_specs=[
            pl.BlockSpec((_BR, D), lambda i: (i, 0)),
            pl.BlockSpec((D, D), lambda i: (0, 0)),
            pl.BlockSpec((1, D), lambda i: (0, 0)),
        ],
        out_specs=pl.BlockSpec((_BR, D), lambda i: (i, 0)),
        out_shape=jax.ShapeDtypeStruct((N_NODES, D), jnp.float32),
    )(x, W_r, b)


def _tc_body(p_ref, cnt_ref, y_ref, wl_ref, o_ref):
    p = jnp.concatenate([p_ref[0], p_ref[1]], axis=-1)
    cnt = cnt_ref[0, :, :1] + cnt_ref[1, :, :1]
    scale = 1.0 / jnp.maximum(cnt, 1.0)
    aggr = p * scale
    dn = (((1,), (1,)), ((), ()))
    o_ref[...] = (
        lax.dot_general(aggr, wl_ref[...], dn, preferred_element_type=jnp.float32)
        + y_ref[...]
    )


def _tc_finish(p, cnt, y, W_l):
    grid = (N_NODES // _BR,)
    return pl.pallas_call(
        _tc_body,
        grid=grid,
        in_specs=[
            pl.BlockSpec((NC, _BR, DH), lambda i: (0, i, 0)),
            pl.BlockSpec((NC, _BR, CNT_W), lambda i: (0, i, 0)),
            pl.BlockSpec((_BR, D), lambda i: (i, 0)),
            pl.BlockSpec((D, D), lambda i: (0, 0)),
        ],
        out_specs=pl.BlockSpec((_BR, D), lambda i: (i, 0)),
        out_shape=jax.ShapeDtypeStruct((N_NODES, D), jnp.float32),
    )(p, cnt, y, W_l)


def kernel(x, edge_index, W_l, W_r, b):
    x2 = x.reshape(NC * N_NODES, DH)
    eidx = (edge_index * jnp.array([[2], [1]], dtype=jnp.int32)).reshape(
        2, NS, N_CHUNKS, CHUNK)
    y = _tc_self(x, W_r, b.reshape(1, D))
    p, cnt_w = _sc_aggregate(x2, eidx)
    return _tc_finish(p, cnt_w, y, W_l)


# speedup vs baseline: 11.7396x; 1.0337x over previous
import functools

import jax
import jax.numpy as jnp
from jax import lax
from jax.experimental import pallas as pl
from jax.experimental.pallas import tpu as pltpu
from jax.experimental.pallas import tpu_sc as plsc

N_NODES = 10000
N_EDGES = 320000
D = 128

NC = 2
NS = 16
LANES = 16

DH = D // NC
EDGES_PER_TILE = N_EDGES // NS
CHUNK = 125
N_CHUNKS = EDGES_PER_TILE // CHUNK
DEPTH = 5
N_GROUPS = N_CHUNKS // DEPTH
N_CHUNKS_HALF = N_CHUNKS // 2
ROWS_PER_TILE = 632
N_PAD = NS * ROWS_PER_TILE
Q_STAGE = 4
QROWS = ROWS_PER_TILE // Q_STAGE
CNT_W = LANES

_mesh = plsc.VectorSubcoreMesh(core_axis_name="c", subcore_axis_name="s")


@functools.partial(
    pl.kernel,
    out_type=(
        jax.ShapeDtypeStruct((NC, N_PAD, DH), jnp.float32),
        jax.ShapeDtypeStruct((NC, N_PAD, CNT_W), jnp.float32),
    ),
    mesh=_mesh,
    compiler_params=pltpu.CompilerParams(use_tc_tiling_on_sc=False),
    scratch_types=(
        pltpu.VMEM_SHARED((N_PAD, DH), jnp.float32),
        pltpu.VMEM_SHARED((N_PAD, CNT_W), jnp.float32),
        pltpu.VMEM((QROWS, DH), jnp.float32),
        pltpu.VMEM((QROWS, CNT_W), jnp.float32),
        pltpu.VMEM((CHUNK, CNT_W), jnp.float32),
        [pltpu.VMEM((2, CHUNK), jnp.int32) for _ in range(DEPTH)],
        [pltpu.VMEM((CHUNK, DH), jnp.float32) for _ in range(DEPTH)],
        [pltpu.SemaphoreType.DMA for _ in range(DEPTH)],
        [pltpu.SemaphoreType.DMA for _ in range(DEPTH)],
        [pltpu.SemaphoreType.DMA for _ in range(DEPTH)],
        [pltpu.SemaphoreType.DMA for _ in range(DEPTH)],
    ),
)
def _sc_aggregate(xt_hbm, eidx_hbm, acc_out, cnt_out,
                  acc_sh, cnt_sh, zrow, zcnt, ones_b,
                  idx_bufs, row_bufs, idx_sems, row_sems, scat_sems, cnt_sems):
    c = lax.axis_index("c")
    s = lax.axis_index("s")

    zero16 = jnp.zeros((LANES,), jnp.float32)
    one16 = jnp.ones((LANES,), jnp.float32)

    def fill_zero(i, carry):
        for j in range(DH // LANES):
            zrow[i, pl.ds(j * LANES, LANES)] = zero16
        zcnt[i, :] = zero16
        return carry

    lax.fori_loop(0, QROWS, fill_zero, 0)

    def fill_one(i, carry):
        ones_b[i, :] = one16
        return carry

    lax.fori_loop(0, CHUNK, fill_one, 0)

    my_src = eidx_hbm.at[0, s]
    my_dst = eidx_hbm.at[1, s]

    def idx_start(chunk, buf, sem):
        pltpu.async_copy(my_src.at[chunk], buf.at[0], sem)
        pltpu.async_copy(my_dst.at[chunk], buf.at[1], sem)

    def idx_wait(chunk, buf, sem):
        pltpu.make_async_copy(my_src.at[chunk], buf.at[0], sem).wait()
        pltpu.make_async_copy(my_dst.at[chunk], buf.at[1], sem).wait()

    row0 = pl.multiple_of(s * ROWS_PER_TILE, 8)
    for u in range(DEPTH):
        idx_start(u, idx_bufs[u], idx_sems[u])
    for q in range(Q_STAGE):
        pltpu.sync_copy(zrow, acc_sh.at[pl.ds(row0 + q * QROWS, QROWS)])
        pltpu.sync_copy(zcnt, cnt_sh.at[pl.ds(row0 + q * QROWS, QROWS)])
    plsc.subcore_barrier()

    x_half = xt_hbm.at[pl.ds(c, 2 * N_NODES - 1)]

    def gather_start(ibuf, buf, sem):
        pltpu.async_copy(x_half.at[ibuf.at[0]], buf, sem)

    def gather_wait(ibuf, buf, sem):
        pltpu.make_async_copy(x_half.at[ibuf.at[0]], buf, sem).wait()

    def do_cnt(chunk):
        return (chunk < N_CHUNKS_HALF) == (c == 0)

    def scat_start(chunk, u):
        pltpu.async_copy(row_bufs[u], acc_sh.at[idx_bufs[u].at[1]],
                         scat_sems[u], add=True)

        @pl.when(do_cnt(chunk))
        def _():
            pltpu.async_copy(ones_b, cnt_sh.at[idx_bufs[u].at[1]],
                             cnt_sems[u], add=True)

    def scat_wait(chunk, u):
        pltpu.make_async_copy(row_bufs[u], acc_sh.at[idx_bufs[u].at[1]],
                              scat_sems[u]).wait()

        @pl.when(do_cnt(chunk))
        def _():
            pltpu.make_async_copy(ones_b, cnt_sh.at[idx_bufs[u].at[1]],
                                  cnt_sems[u]).wait()

    for u in range(DEPTH):
        idx_wait(u, idx_bufs[u], idx_sems[u])
        gather_start(idx_bufs[u], row_bufs[u], row_sems[u])

    def step(k, carry):
        i = DEPTH * k
        for u in range(DEPTH):
            gather_wait(idx_bufs[u], row_bufs[u], row_sems[u])
            scat_start(i + u, u)

        for u in range(DEPTH):
            @pl.when(k < N_GROUPS - 1)
            def _():
                scat_wait(i + u, u)
                idx_start(i + u + DEPTH, idx_bufs[u], idx_sems[u])
                idx_wait(i + u + DEPTH, idx_bufs[u], idx_sems[u])
                gather_start(idx_bufs[u], row_bufs[u], row_sems[u])

        return carry

    lax.fori_loop(0, N_GROUPS, step, 0)

    for u in range(DEPTH):
        scat_wait(N_CHUNKS - DEPTH + u, u)
    plsc.subcore_barrier()

    for q in range(Q_STAGE):
        r = row0 + q * QROWS
        pltpu.sync_copy(acc_sh.at[pl.ds(r, QROWS)], zrow)
        pltpu.sync_copy(zrow, acc_out.at[c, pl.ds(r, QROWS)])
        pltpu.sync_copy(cnt_sh.at[pl.ds(r, QROWS)], zcnt)
        pltpu.sync_copy(zcnt, cnt_out.at[c, pl.ds(r, QROWS)])


_BR = 400


def _tc_self_body(x_ref, wr_ref, b_ref, y_ref):
    dn = (((1,), (1,)), ((), ()))
    y_ref[...] = (
        lax.dot_general(x_ref[...], wr_ref[...], dn, preferred_element_type=jnp.float32)
        + b_ref[...]
    )


def _tc_self(x, W_r, b):
    return pl.pallas_call(
        _tc_self_body,
        grid=(N_NODES // _BR,),
        in_specs=[
            pl.BlockSpec((_BR, D), lambda i: (i, 0)),
            pl.BlockSpec((D, D), lambda i: (0, 0)),
            pl.BlockSpec((1, D), lambda i: (0, 0)),
        ],
        out_specs=pl.BlockSpec((_BR, D), lambda i: (i, 0)),
        out_shape=jax.ShapeDtypeStruct((N_NODES, D), jnp.float32),
    )(x, W_r, b)


def _tc_body(p_ref, cnt_ref, y_ref, wl_ref, o_ref):
    p = jnp.concatenate([p_ref[0], p_ref[1]], axis=-1)
    cnt = cnt_ref[0, :, :1] + cnt_ref[1, :, :1]
    scale = 1.0 / jnp.maximum(cnt, 1.0)
    aggr = p * scale
    dn = (((1,), (1,)), ((), ()))
    o_ref[...] = (
        lax.dot_general(aggr, wl_ref[...], dn, preferred_element_type=jnp.float32)
        + y_ref[...]
    )


def _tc_finish(p, cnt, y, W_l):
    grid = (N_NODES // _BR,)
    return pl.pallas_call(
        _tc_body,
        grid=grid,
        in_specs=[
            pl.BlockSpec((NC, _BR, DH), lambda i: (0, i, 0)),
            pl.BlockSpec((NC, _BR, CNT_W), lambda i: (0, i, 0)),
            pl.BlockSpec((_BR, D), lambda i: (i, 0)),
            pl.BlockSpec((D, D), lambda i: (0, 0)),
        ],
        out_specs=pl.BlockSpec((_BR, D), lambda i: (i, 0)),
        out_shape=jax.ShapeDtypeStruct((N_NODES, D), jnp.float32),
    )(p, cnt, y, W_l)


def kernel(x, edge_index, W_l, W_r, b):
    x2 = x.reshape(NC * N_NODES, DH)
    eidx = (edge_index * jnp.array([[2], [1]], dtype=jnp.int32)).reshape(
        2, NS, N_CHUNKS, CHUNK)
    y = _tc_self(x, W_r, b.reshape(1, D))
    p, cnt_w = _sc_aggregate(x2, eidx)
    return _tc_finish(p, cnt_w, y, W_l)
